# baseline (speedup 1.0000x reference)
import functools

import jax
import jax.numpy as jnp
from jax import lax
from jax.experimental import pallas as pl
from jax.experimental.pallas import tpu as pltpu
from jax.experimental.pallas import tpu_sc as plsc

IGNORE_LABEL = 255
THRESH = 0.35667494393873245

NC, NS, L = 2, 16, 16
NW = NC * NS

HB = 1024
INV_W = HB / THRESH
CH = 8192

N_PIX = 8 * 512 * 512
PER_W = N_PIX // NW
N_CHUNKS = PER_W // CH
N_MIN = float(max(N_PIX // 16, 1))


def _tc_loss_body(p_ref, t_ref, o_ref):
    t = t_ref[0]
    m = p_ref[0, 0]
    for c in range(1, 19):
        m = jnp.maximum(m, p_ref[0, c])
    s = jnp.zeros_like(m)
    xt = jnp.zeros_like(m)
    for c in range(19):
        xc = p_ref[0, c]
        s = s + jnp.exp(xc - m)
        xt = jnp.where(t == c, xc, xt)
    loss = m + jnp.log(s) - xt
    o_ref[...] = jnp.where(t == IGNORE_LABEL, 0.0, loss).reshape(-1)


def _tc_loss(preds, targets, b0, nb):
    B, C, H, W = preds.shape
    return pl.pallas_call(
        _tc_loss_body,
        grid=(nb,),
        in_specs=[
            pl.BlockSpec((1, C, H, W), lambda b: (b0 + b, 0, 0, 0)),
            pl.BlockSpec((1, H, W), lambda b: (b0 + b, 0, 0)),
        ],
        out_specs=pl.BlockSpec((H * W,), lambda b: (b,)),
        out_shape=jax.ShapeDtypeStruct((nb * H * W,), jnp.float32),
        compiler_params=pltpu.CompilerParams(
            dimension_semantics=("arbitrary",),
        ),
    )(preds, targets)


def _sc_hard_body(n_chunks, loss_hbm, hard_out, buf, obuf):
    wid = lax.axis_index("s") * NC + lax.axis_index("c")
    base = wid * (n_chunks * CH)

    def gbody(g, carry):
        hc, hs = carry
        for u in range(4):
            v = buf[pl.ds((g * 4 + u) * L, L)]
            hard = v > THRESH
            hc = hc + jnp.where(hard, 1.0, 0.0)
            hs = hs + jnp.where(hard, v, 0.0)
        return hc, hs

    hc = jnp.zeros((L,), jnp.float32)
    hs = jnp.zeros((L,), jnp.float32)
    for chunk in range(n_chunks):
        pltpu.sync_copy(loss_hbm.at[pl.ds(base + chunk * CH, CH)], buf)
        hc, hs = lax.fori_loop(0, CH // (4 * L), gbody, (hc, hs))

    obuf[0] = hc
    obuf[1] = hs
    pltpu.sync_copy(obuf, hard_out.at[wid])


def _sc_hard(loss_flat):
    n_chunks = loss_flat.shape[0] // (NW * CH)
    mesh = plsc.VectorSubcoreMesh(core_axis_name="c", subcore_axis_name="s")
    kern = pl.kernel(
        functools.partial(_sc_hard_body, n_chunks),
        mesh=mesh,
        out_type=jax.ShapeDtypeStruct((NW, 2, L), jnp.float32),
        scratch_types=[
            pltpu.VMEM((CH,), jnp.float32),
            pltpu.VMEM((2, L), jnp.float32),
        ],
        compiler_params=pltpu.CompilerParams(needs_layout_passes=False),
    )
    return kern(loss_flat)


def _sc_mine_body(loss_hbm, cnt_out, sum_out, hard_out,
                  buf, cnts, sums, rcnt, rsum, hbuf):
    wid = lax.axis_index("s") * NC + lax.axis_index("c")
    base = wid * PER_W

    zero16 = jnp.zeros((L,), jnp.float32)

    def zbody(i, _):
        cnts[pl.ds(i * L, L)] = zero16
        sums[pl.ds(i * L, L)] = zero16
        return 0

    lax.fori_loop(0, HB, zbody, 0)

    lanes = lax.iota(jnp.int32, L) * HB
    ones = jnp.ones((L,), jnp.float32)

    def gbody(g, carry):
        hc, hs = carry
        v = buf[pl.ds(g * L, L)]
        hard = v > THRESH
        hc = hc + jnp.where(hard, 1.0, 0.0)
        hs = hs + jnp.where(hard, v, 0.0)
        b = (v * INV_W).astype(jnp.int32)
        b = jnp.minimum(b, HB - 1)
        fidx = lanes + b
        easy = jnp.logical_not(hard)
        plsc.addupdate_scatter(cnts, [fidx], ones, mask=easy)
        plsc.addupdate_scatter(sums, [fidx], v, mask=easy)
        return hc, hs

    hc, hs = zero16, zero16
    for chunk in range(N_CHUNKS):
        pltpu.sync_copy(loss_hbm.at[pl.ds(base + chunk * CH, CH)], buf)
        hc, hs = lax.fori_loop(0, CH // L, gbody, (hc, hs))

    def rbody(j, _):
        ac = zero16
        asm = zero16
        for l in range(L):
            ac = ac + cnts[pl.ds(l * HB + j * L, L)]
            asm = asm + sums[pl.ds(l * HB + j * L, L)]
        rcnt[pl.ds(j * L, L)] = ac
        rsum[pl.ds(j * L, L)] = asm
        return 0

    lax.fori_loop(0, HB // L, rbody, 0)

    hbuf[0] = hc
    hbuf[1] = hs
    pltpu.sync_copy(rcnt, cnt_out.at[wid])
    pltpu.sync_copy(rsum, sum_out.at[wid])
    pltpu.sync_copy(hbuf, hard_out.at[wid])


def _sc_mine(loss_flat):
    mesh = plsc.VectorSubcoreMesh(core_axis_name="c", subcore_axis_name="s")
    kern = pl.kernel(
        _sc_mine_body,
        mesh=mesh,
        out_type=[
            jax.ShapeDtypeStruct((NW, HB), jnp.float32),
            jax.ShapeDtypeStruct((NW, HB), jnp.float32),
            jax.ShapeDtypeStruct((NW, 2, L), jnp.float32),
        ],
        scratch_types=[
            pltpu.VMEM((CH,), jnp.float32),
            pltpu.VMEM((L * HB,), jnp.float32),
            pltpu.VMEM((L * HB,), jnp.float32),
            pltpu.VMEM((HB,), jnp.float32),
            pltpu.VMEM((HB,), jnp.float32),
            pltpu.VMEM((2, L), jnp.float32),
        ],
        compiler_params=pltpu.CompilerParams(needs_layout_passes=False),
    )
    return kern(loss_flat)


def _topk_mean(cnt_hist, sum_hist, hard):
    cnt_b = jnp.sum(cnt_hist, axis=0)
    sum_b = jnp.sum(sum_hist, axis=0)
    h = jnp.sum(hard, axis=(0, 2))
    n_hard, sum_hard = h[0], h[1]
    cc = jnp.cumsum(cnt_b[::-1])[::-1]
    above = cc - cnt_b
    need = N_MIN - n_hard
    r = jnp.clip(need - above, 0.0, cnt_b)
    bin_mean = sum_b / jnp.maximum(cnt_b, 1.0)
    return (sum_hard + jnp.sum(r * bin_mean)) / N_MIN


def _finish(cnt_hist, sum_hist, hard):
    h = jnp.sum(hard, axis=(0, 2))
    n_hard = h[0]
    hard_mean = h[1] / jnp.maximum(n_hard, 1.0)
    return jnp.where(n_hard < N_MIN, _topk_mean(cnt_hist, sum_hist, hard),
                     hard_mean)


S = 4
BS = 8 // S


def kernel(preds, targets):
    t32 = targets.astype(jnp.int32)
    parts = [_tc_loss(preds, t32, i * BS, BS) for i in range(S)]
    hstats = [_sc_hard(p) for p in parts]
    h = jnp.sum(jnp.stack(hstats), axis=(0, 1, 3))
    n_hard, sum_hard = h[0], h[1]
    hard_mean = sum_hard / jnp.maximum(n_hard, 1.0)

    def rare(_):
        cnt_hist, sum_hist, hard = _sc_mine(jnp.concatenate(parts))
        return _topk_mean(cnt_hist, sum_hist, hard)

    def common(_):
        return hard_mean

    return lax.cond(n_hard < N_MIN, rare, common, None)


# speedup vs baseline: 1.1031x; 1.1031x over previous
import functools

import jax
import jax.numpy as jnp
from jax import lax
from jax.experimental import pallas as pl
from jax.experimental.pallas import tpu as pltpu
from jax.experimental.pallas import tpu_sc as plsc

IGNORE_LABEL = 255
THRESH = 0.35667494393873245

NC, NS, L = 2, 16, 16
NW = NC * NS

HB = 1024
INV_W = HB / THRESH
CH = 8192

N_PIX = 8 * 512 * 512
PER_W = N_PIX // NW
N_CHUNKS = PER_W // CH
N_MIN = float(max(N_PIX // 16, 1))


def _tc_loss_body(p_ref, t_ref, o_ref):
    t = t_ref[0]
    m = p_ref[0, 0]
    for c in range(1, 19):
        m = jnp.maximum(m, p_ref[0, c])
    s = jnp.zeros_like(m)
    xt = jnp.zeros_like(m)
    for c in range(19):
        xc = p_ref[0, c]
        s = s + jnp.exp(xc - m)
        xt = jnp.where(t == c, xc, xt)
    loss = m + jnp.log(s) - xt
    o_ref[...] = jnp.where(t == IGNORE_LABEL, 0.0, loss).reshape(-1)


def _tc_loss(preds, targets, b0, nb):
    B, C, H, W = preds.shape
    return pl.pallas_call(
        _tc_loss_body,
        grid=(nb,),
        in_specs=[
            pl.BlockSpec((1, C, H, W), lambda b: (b0 + b, 0, 0, 0)),
            pl.BlockSpec((1, H, W), lambda b: (b0 + b, 0, 0)),
        ],
        out_specs=pl.BlockSpec((H * W,), lambda b: (b,)),
        out_shape=jax.ShapeDtypeStruct((nb * H * W,), jnp.float32),
        compiler_params=pltpu.CompilerParams(
            dimension_semantics=("arbitrary",),
        ),
    )(preds, targets)


def _sc_hard_body(n_chunks, loss_hbm, hard_out, buf, obuf):
    wid = lax.axis_index("s") * NC + lax.axis_index("c")
    base = wid * (n_chunks * CH)

    def gbody(g, carry):
        hc, hs = carry
        for u in range(4):
            v = buf[pl.ds((g * 4 + u) * L, L)]
            hard = v > THRESH
            hc = hc + jnp.where(hard, 1.0, 0.0)
            hs = hs + jnp.where(hard, v, 0.0)
        return hc, hs

    hc = jnp.zeros((L,), jnp.float32)
    hs = jnp.zeros((L,), jnp.float32)
    for chunk in range(n_chunks):
        pltpu.sync_copy(loss_hbm.at[pl.ds(base + chunk * CH, CH)], buf)
        hc, hs = lax.fori_loop(0, CH // (4 * L), gbody, (hc, hs))

    obuf[0] = hc
    obuf[1] = hs
    pltpu.sync_copy(obuf, hard_out.at[wid])


def _sc_hard(loss_flat):
    n_chunks = loss_flat.shape[0] // (NW * CH)
    mesh = plsc.VectorSubcoreMesh(core_axis_name="c", subcore_axis_name="s")
    kern = pl.kernel(
        functools.partial(_sc_hard_body, n_chunks),
        mesh=mesh,
        out_type=jax.ShapeDtypeStruct((NW, 2, L), jnp.float32),
        scratch_types=[
            pltpu.VMEM((CH,), jnp.float32),
            pltpu.VMEM((2, L), jnp.float32),
        ],
        compiler_params=pltpu.CompilerParams(needs_layout_passes=False),
    )
    return kern(loss_flat)


def _sc_mine_body(loss_hbm, cnt_out, sum_out, hard_out,
                  buf, cnts, sums, rcnt, rsum, hbuf):
    wid = lax.axis_index("s") * NC + lax.axis_index("c")
    base = wid * PER_W

    zero16 = jnp.zeros((L,), jnp.float32)

    def zbody(i, _):
        cnts[pl.ds(i * L, L)] = zero16
        sums[pl.ds(i * L, L)] = zero16
        return 0

    lax.fori_loop(0, HB, zbody, 0)

    lanes = lax.iota(jnp.int32, L) * HB
    ones = jnp.ones((L,), jnp.float32)

    def gbody(g, carry):
        hc, hs = carry
        v = buf[pl.ds(g * L, L)]
        hard = v > THRESH
        hc = hc + jnp.where(hard, 1.0, 0.0)
        hs = hs + jnp.where(hard, v, 0.0)
        b = (v * INV_W).astype(jnp.int32)
        b = jnp.minimum(b, HB - 1)
        fidx = lanes + b
        easy = jnp.logical_not(hard)
        plsc.addupdate_scatter(cnts, [fidx], ones, mask=easy)
        plsc.addupdate_scatter(sums, [fidx], v, mask=easy)
        return hc, hs

    hc, hs = zero16, zero16
    for chunk in range(N_CHUNKS):
        pltpu.sync_copy(loss_hbm.at[pl.ds(base + chunk * CH, CH)], buf)
        hc, hs = lax.fori_loop(0, CH // L, gbody, (hc, hs))

    def rbody(j, _):
        ac = zero16
        asm = zero16
        for l in range(L):
            ac = ac + cnts[pl.ds(l * HB + j * L, L)]
            asm = asm + sums[pl.ds(l * HB + j * L, L)]
        rcnt[pl.ds(j * L, L)] = ac
        rsum[pl.ds(j * L, L)] = asm
        return 0

    lax.fori_loop(0, HB // L, rbody, 0)

    hbuf[0] = hc
    hbuf[1] = hs
    pltpu.sync_copy(rcnt, cnt_out.at[wid])
    pltpu.sync_copy(rsum, sum_out.at[wid])
    pltpu.sync_copy(hbuf, hard_out.at[wid])


def _sc_mine(loss_flat):
    mesh = plsc.VectorSubcoreMesh(core_axis_name="c", subcore_axis_name="s")
    kern = pl.kernel(
        _sc_mine_body,
        mesh=mesh,
        out_type=[
            jax.ShapeDtypeStruct((NW, HB), jnp.float32),
            jax.ShapeDtypeStruct((NW, HB), jnp.float32),
            jax.ShapeDtypeStruct((NW, 2, L), jnp.float32),
        ],
        scratch_types=[
            pltpu.VMEM((CH,), jnp.float32),
            pltpu.VMEM((L * HB,), jnp.float32),
            pltpu.VMEM((L * HB,), jnp.float32),
            pltpu.VMEM((HB,), jnp.float32),
            pltpu.VMEM((HB,), jnp.float32),
            pltpu.VMEM((2, L), jnp.float32),
        ],
        compiler_params=pltpu.CompilerParams(needs_layout_passes=False),
    )
    return kern(loss_flat)


def _topk_mean(cnt_hist, sum_hist, hard):
    cnt_b = jnp.sum(cnt_hist, axis=0)
    sum_b = jnp.sum(sum_hist, axis=0)
    h = jnp.sum(hard, axis=(0, 2))
    n_hard, sum_hard = h[0], h[1]
    cc = jnp.cumsum(cnt_b[::-1])[::-1]
    above = cc - cnt_b
    need = N_MIN - n_hard
    r = jnp.clip(need - above, 0.0, cnt_b)
    bin_mean = sum_b / jnp.maximum(cnt_b, 1.0)
    return (sum_hard + jnp.sum(r * bin_mean)) / N_MIN


def _finish(cnt_hist, sum_hist, hard):
    h = jnp.sum(hard, axis=(0, 2))
    n_hard = h[0]
    hard_mean = h[1] / jnp.maximum(n_hard, 1.0)
    return jnp.where(n_hard < N_MIN, _topk_mean(cnt_hist, sum_hist, hard),
                     hard_mean)


S = 2
BS = 8 // S


def kernel(preds, targets):
    t32 = targets.astype(jnp.int32)
    parts = [_tc_loss(preds, t32, i * BS, BS) for i in range(S)]
    hstats = [_sc_hard(p) for p in parts]
    h = jnp.sum(jnp.stack(hstats), axis=(0, 1, 3))
    n_hard, sum_hard = h[0], h[1]
    hard_mean = sum_hard / jnp.maximum(n_hard, 1.0)

    def rare(_):
        cnt_hist, sum_hist, hard = _sc_mine(jnp.concatenate(parts))
        return _topk_mean(cnt_hist, sum_hist, hard)

    def common(_):
        return hard_mean

    return lax.cond(n_hard < N_MIN, rare, common, None)


# speedup vs baseline: 1.1244x; 1.0193x over previous
import functools

import jax
import jax.numpy as jnp
from jax import lax
from jax.experimental import pallas as pl
from jax.experimental.pallas import tpu as pltpu
from jax.experimental.pallas import tpu_sc as plsc

IGNORE_LABEL = 255
THRESH = 0.35667494393873245

NC, NS, L = 2, 16, 16
NW = NC * NS

HB = 1024
INV_W = HB / THRESH
CH = 8192

N_PIX = 8 * 512 * 512
PER_W = N_PIX // NW
N_CHUNKS = PER_W // CH
N_MIN = float(max(N_PIX // 16, 1))


def _tc_loss_body(p_ref, t_ref, o_ref):
    t = t_ref[0]
    m = p_ref[0, 0]
    for c in range(1, 19):
        m = jnp.maximum(m, p_ref[0, c])
    s = jnp.zeros_like(m)
    xt = jnp.zeros_like(m)
    for c in range(19):
        xc = p_ref[0, c]
        s = s + jnp.exp(xc - m)
        xt = jnp.where(t == c, xc, xt)
    loss = m + jnp.log(s) - xt
    o_ref[...] = jnp.where(t == IGNORE_LABEL, 0.0, loss).reshape(-1)


def _tc_loss(preds, targets, b0, nb):
    B, C, H, W = preds.shape
    return pl.pallas_call(
        _tc_loss_body,
        grid=(nb,),
        in_specs=[
            pl.BlockSpec((1, C, H, W), lambda b: (b0 + b, 0, 0, 0)),
            pl.BlockSpec((1, H, W), lambda b: (b0 + b, 0, 0)),
        ],
        out_specs=pl.BlockSpec((H * W,), lambda b: (b,)),
        out_shape=jax.ShapeDtypeStruct((nb * H * W,), jnp.float32),
        compiler_params=pltpu.CompilerParams(
            dimension_semantics=("arbitrary",),
        ),
    )(preds, targets)


def _sc_hard_body(n_chunks, loss_hbm, hard_out, buf, obuf):
    wid = lax.axis_index("s") * NC + lax.axis_index("c")
    base = wid * (n_chunks * CH)

    def gbody(g, carry):
        hc, hs = carry
        for u in range(4):
            v = buf[pl.ds((g * 4 + u) * L, L)]
            hard = v > THRESH
            hc = hc + jnp.where(hard, 1.0, 0.0)
            hs = hs + jnp.where(hard, v, 0.0)
        return hc, hs

    hc = jnp.zeros((L,), jnp.float32)
    hs = jnp.zeros((L,), jnp.float32)
    for chunk in range(n_chunks):
        pltpu.sync_copy(loss_hbm.at[pl.ds(base + chunk * CH, CH)], buf)
        hc, hs = lax.fori_loop(0, CH // (4 * L), gbody, (hc, hs))

    obuf[0] = hc
    obuf[1] = hs
    pltpu.sync_copy(obuf, hard_out.at[wid])


def _sc_hard(loss_flat):
    n_chunks = loss_flat.shape[0] // (NW * CH)
    mesh = plsc.VectorSubcoreMesh(core_axis_name="c", subcore_axis_name="s")
    kern = pl.kernel(
        functools.partial(_sc_hard_body, n_chunks),
        mesh=mesh,
        out_type=jax.ShapeDtypeStruct((NW, 2, L), jnp.float32),
        scratch_types=[
            pltpu.VMEM((CH,), jnp.float32),
            pltpu.VMEM((2, L), jnp.float32),
        ],
        compiler_params=pltpu.CompilerParams(needs_layout_passes=False),
    )
    return kern(loss_flat)


def _sc_mine_body(loss_hbm, cnt_out, sum_out, hard_out,
                  buf, cnts, sums, rcnt, rsum, hbuf):
    wid = lax.axis_index("s") * NC + lax.axis_index("c")
    base = wid * PER_W

    zero16 = jnp.zeros((L,), jnp.float32)

    def zbody(i, _):
        cnts[pl.ds(i * L, L)] = zero16
        sums[pl.ds(i * L, L)] = zero16
        return 0

    lax.fori_loop(0, HB, zbody, 0)

    lanes = lax.iota(jnp.int32, L) * HB
    ones = jnp.ones((L,), jnp.float32)

    def gbody(g, carry):
        hc, hs = carry
        v = buf[pl.ds(g * L, L)]
        hard = v > THRESH
        hc = hc + jnp.where(hard, 1.0, 0.0)
        hs = hs + jnp.where(hard, v, 0.0)
        b = (v * INV_W).astype(jnp.int32)
        b = jnp.minimum(b, HB - 1)
        fidx = lanes + b
        easy = jnp.logical_not(hard)
        plsc.addupdate_scatter(cnts, [fidx], ones, mask=easy)
        plsc.addupdate_scatter(sums, [fidx], v, mask=easy)
        return hc, hs

    hc, hs = zero16, zero16
    for chunk in range(N_CHUNKS):
        pltpu.sync_copy(loss_hbm.at[pl.ds(base + chunk * CH, CH)], buf)
        hc, hs = lax.fori_loop(0, CH // L, gbody, (hc, hs))

    def rbody(j, _):
        ac = zero16
        asm = zero16
        for l in range(L):
            ac = ac + cnts[pl.ds(l * HB + j * L, L)]
            asm = asm + sums[pl.ds(l * HB + j * L, L)]
        rcnt[pl.ds(j * L, L)] = ac
        rsum[pl.ds(j * L, L)] = asm
        return 0

    lax.fori_loop(0, HB // L, rbody, 0)

    hbuf[0] = hc
    hbuf[1] = hs
    pltpu.sync_copy(rcnt, cnt_out.at[wid])
    pltpu.sync_copy(rsum, sum_out.at[wid])
    pltpu.sync_copy(hbuf, hard_out.at[wid])


def _sc_mine(loss_flat):
    mesh = plsc.VectorSubcoreMesh(core_axis_name="c", subcore_axis_name="s")
    kern = pl.kernel(
        _sc_mine_body,
        mesh=mesh,
        out_type=[
            jax.ShapeDtypeStruct((NW, HB), jnp.float32),
            jax.ShapeDtypeStruct((NW, HB), jnp.float32),
            jax.ShapeDtypeStruct((NW, 2, L), jnp.float32),
        ],
        scratch_types=[
            pltpu.VMEM((CH,), jnp.float32),
            pltpu.VMEM((L * HB,), jnp.float32),
            pltpu.VMEM((L * HB,), jnp.float32),
            pltpu.VMEM((HB,), jnp.float32),
            pltpu.VMEM((HB,), jnp.float32),
            pltpu.VMEM((2, L), jnp.float32),
        ],
        compiler_params=pltpu.CompilerParams(needs_layout_passes=False),
    )
    return kern(loss_flat)


def _topk_mean(cnt_hist, sum_hist, hard):
    cnt_b = jnp.sum(cnt_hist, axis=0)
    sum_b = jnp.sum(sum_hist, axis=0)
    h = jnp.sum(hard, axis=(0, 2))
    n_hard, sum_hard = h[0], h[1]
    cc = jnp.cumsum(cnt_b[::-1])[::-1]
    above = cc - cnt_b
    need = N_MIN - n_hard
    r = jnp.clip(need - above, 0.0, cnt_b)
    bin_mean = sum_b / jnp.maximum(cnt_b, 1.0)
    return (sum_hard + jnp.sum(r * bin_mean)) / N_MIN


def _finish(cnt_hist, sum_hist, hard):
    h = jnp.sum(hard, axis=(0, 2))
    n_hard = h[0]
    hard_mean = h[1] / jnp.maximum(n_hard, 1.0)
    return jnp.where(n_hard < N_MIN, _topk_mean(cnt_hist, sum_hist, hard),
                     hard_mean)


S = 1
BS = 8 // S


def kernel(preds, targets):
    t32 = targets.astype(jnp.int32)
    parts = [_tc_loss(preds, t32, i * BS, BS) for i in range(S)]
    hstats = [_sc_hard(p) for p in parts]
    h = jnp.sum(jnp.stack(hstats), axis=(0, 1, 3))
    n_hard, sum_hard = h[0], h[1]
    hard_mean = sum_hard / jnp.maximum(n_hard, 1.0)

    def rare(_):
        cnt_hist, sum_hist, hard = _sc_mine(jnp.concatenate(parts))
        return _topk_mean(cnt_hist, sum_hist, hard)

    def common(_):
        return hard_mean

    return lax.cond(n_hard < N_MIN, rare, common, None)


# speedup vs baseline: 1.1963x; 1.0639x over previous
import functools

import jax
import jax.numpy as jnp
from jax import lax
from jax.experimental import pallas as pl
from jax.experimental.pallas import tpu as pltpu
from jax.experimental.pallas import tpu_sc as plsc

IGNORE_LABEL = 255
THRESH = 0.35667494393873245

NC, NS, L = 2, 16, 16
NW = NC * NS

HB = 1024
INV_W = HB / THRESH
CH = 8192

N_PIX = 8 * 512 * 512
PER_W = N_PIX // NW
N_CHUNKS = PER_W // CH
N_MIN = float(max(N_PIX // 16, 1))


def _tc_loss_body(p_ref, t_ref, o_ref):
    t = t_ref[0]
    m = p_ref[0, 0]
    for c in range(1, 19):
        m = jnp.maximum(m, p_ref[0, c])
    s = jnp.zeros_like(m)
    xt = jnp.zeros_like(m)
    for c in range(19):
        xc = p_ref[0, c]
        s = s + jnp.exp(xc - m)
        xt = jnp.where(t == c, xc, xt)
    loss = m + jnp.log(s) - xt
    o_ref[...] = jnp.where(t == IGNORE_LABEL, 0.0, loss).reshape(-1)


def _tc_loss(preds, targets, b0, nb):
    B, C, H, W = preds.shape
    return pl.pallas_call(
        _tc_loss_body,
        grid=(nb,),
        in_specs=[
            pl.BlockSpec((1, C, H, W), lambda b: (b0 + b, 0, 0, 0)),
            pl.BlockSpec((1, H, W), lambda b: (b0 + b, 0, 0)),
        ],
        out_specs=pl.BlockSpec((H * W,), lambda b: (b,)),
        out_shape=jax.ShapeDtypeStruct((nb * H * W,), jnp.float32),
        compiler_params=pltpu.CompilerParams(
            dimension_semantics=("arbitrary",),
        ),
    )(preds, targets)


def _sc_hard_body(n_chunks, loss_hbm, hard_out, buf0, buf1, obuf, sem0, sem1):
    wid = lax.axis_index("s") * NC + lax.axis_index("c")
    base = wid * (n_chunks * CH)
    bufs = (buf0, buf1)
    sems = (sem0, sem1)

    def copy(chunk):
        return pltpu.make_async_copy(
            loss_hbm.at[pl.ds(base + chunk * CH, CH)],
            bufs[chunk % 2], sems[chunk % 2])

    def gbody(buf):
        def body(g, carry):
            hc, hs = carry
            for u in range(4):
                v = buf[pl.ds((g * 4 + u) * L, L)]
                hard = v > THRESH
                hc = hc + jnp.where(hard, 1.0, 0.0)
                hs = hs + jnp.where(hard, v, 0.0)
            return hc, hs
        return body

    hc = jnp.zeros((L,), jnp.float32)
    hs = jnp.zeros((L,), jnp.float32)
    copy(0).start()
    for chunk in range(n_chunks):
        if chunk + 1 < n_chunks:
            copy(chunk + 1).start()
        copy(chunk).wait()
        hc, hs = lax.fori_loop(0, CH // (4 * L), gbody(bufs[chunk % 2]),
                               (hc, hs))

    obuf[0] = hc
    obuf[1] = hs
    pltpu.sync_copy(obuf, hard_out.at[wid])


def _sc_hard(loss_flat):
    n_chunks = loss_flat.shape[0] // (NW * CH)
    mesh = plsc.VectorSubcoreMesh(core_axis_name="c", subcore_axis_name="s")
    kern = pl.kernel(
        functools.partial(_sc_hard_body, n_chunks),
        mesh=mesh,
        out_type=jax.ShapeDtypeStruct((NW, 2, L), jnp.float32),
        scratch_types=[
            pltpu.VMEM((CH,), jnp.float32),
            pltpu.VMEM((CH,), jnp.float32),
            pltpu.VMEM((2, L), jnp.float32),
            pltpu.SemaphoreType.DMA,
            pltpu.SemaphoreType.DMA,
        ],
        compiler_params=pltpu.CompilerParams(needs_layout_passes=False),
    )
    return kern(loss_flat)


def _sc_mine_body(loss_hbm, cnt_out, sum_out, hard_out,
                  buf, cnts, sums, rcnt, rsum, hbuf):
    wid = lax.axis_index("s") * NC + lax.axis_index("c")
    base = wid * PER_W

    zero16 = jnp.zeros((L,), jnp.float32)

    def zbody(i, _):
        cnts[pl.ds(i * L, L)] = zero16
        sums[pl.ds(i * L, L)] = zero16
        return 0

    lax.fori_loop(0, HB, zbody, 0)

    lanes = lax.iota(jnp.int32, L) * HB
    ones = jnp.ones((L,), jnp.float32)

    def gbody(g, carry):
        hc, hs = carry
        v = buf[pl.ds(g * L, L)]
        hard = v > THRESH
        hc = hc + jnp.where(hard, 1.0, 0.0)
        hs = hs + jnp.where(hard, v, 0.0)
        b = (v * INV_W).astype(jnp.int32)
        b = jnp.minimum(b, HB - 1)
        fidx = lanes + b
        easy = jnp.logical_not(hard)
        plsc.addupdate_scatter(cnts, [fidx], ones, mask=easy)
        plsc.addupdate_scatter(sums, [fidx], v, mask=easy)
        return hc, hs

    hc, hs = zero16, zero16
    for chunk in range(N_CHUNKS):
        pltpu.sync_copy(loss_hbm.at[pl.ds(base + chunk * CH, CH)], buf)
        hc, hs = lax.fori_loop(0, CH // L, gbody, (hc, hs))

    def rbody(j, _):
        ac = zero16
        asm = zero16
        for l in range(L):
            ac = ac + cnts[pl.ds(l * HB + j * L, L)]
            asm = asm + sums[pl.ds(l * HB + j * L, L)]
        rcnt[pl.ds(j * L, L)] = ac
        rsum[pl.ds(j * L, L)] = asm
        return 0

    lax.fori_loop(0, HB // L, rbody, 0)

    hbuf[0] = hc
    hbuf[1] = hs
    pltpu.sync_copy(rcnt, cnt_out.at[wid])
    pltpu.sync_copy(rsum, sum_out.at[wid])
    pltpu.sync_copy(hbuf, hard_out.at[wid])


def _sc_mine(loss_flat):
    mesh = plsc.VectorSubcoreMesh(core_axis_name="c", subcore_axis_name="s")
    kern = pl.kernel(
        _sc_mine_body,
        mesh=mesh,
        out_type=[
            jax.ShapeDtypeStruct((NW, HB), jnp.float32),
            jax.ShapeDtypeStruct((NW, HB), jnp.float32),
            jax.ShapeDtypeStruct((NW, 2, L), jnp.float32),
        ],
        scratch_types=[
            pltpu.VMEM((CH,), jnp.float32),
            pltpu.VMEM((L * HB,), jnp.float32),
            pltpu.VMEM((L * HB,), jnp.float32),
            pltpu.VMEM((HB,), jnp.float32),
            pltpu.VMEM((HB,), jnp.float32),
            pltpu.VMEM((2, L), jnp.float32),
        ],
        compiler_params=pltpu.CompilerParams(needs_layout_passes=False),
    )
    return kern(loss_flat)


def _topk_mean(cnt_hist, sum_hist, hard):
    cnt_b = jnp.sum(cnt_hist, axis=0)
    sum_b = jnp.sum(sum_hist, axis=0)
    h = jnp.sum(hard, axis=(0, 2))
    n_hard, sum_hard = h[0], h[1]
    cc = jnp.cumsum(cnt_b[::-1])[::-1]
    above = cc - cnt_b
    need = N_MIN - n_hard
    r = jnp.clip(need - above, 0.0, cnt_b)
    bin_mean = sum_b / jnp.maximum(cnt_b, 1.0)
    return (sum_hard + jnp.sum(r * bin_mean)) / N_MIN


def _finish(cnt_hist, sum_hist, hard):
    h = jnp.sum(hard, axis=(0, 2))
    n_hard = h[0]
    hard_mean = h[1] / jnp.maximum(n_hard, 1.0)
    return jnp.where(n_hard < N_MIN, _topk_mean(cnt_hist, sum_hist, hard),
                     hard_mean)


S = 1
BS = 8 // S


def kernel(preds, targets):
    t32 = targets.astype(jnp.int32)
    parts = [_tc_loss(preds, t32, i * BS, BS) for i in range(S)]
    hstats = [_sc_hard(p) for p in parts]
    h = jnp.sum(jnp.stack(hstats), axis=(0, 1, 3))
    n_hard, sum_hard = h[0], h[1]
    hard_mean = sum_hard / jnp.maximum(n_hard, 1.0)

    def rare(_):
        cnt_hist, sum_hist, hard = _sc_mine(jnp.concatenate(parts))
        return _topk_mean(cnt_hist, sum_hist, hard)

    def common(_):
        return hard_mean

    return lax.cond(n_hard < N_MIN, rare, common, None)


# speedup vs baseline: 1.2014x; 1.0042x over previous
import functools

import jax
import jax.numpy as jnp
from jax import lax
from jax.experimental import pallas as pl
from jax.experimental.pallas import tpu as pltpu
from jax.experimental.pallas import tpu_sc as plsc

IGNORE_LABEL = 255
THRESH = 0.35667494393873245

NC, NS, L = 2, 16, 16
NW = NC * NS

HB = 1024
INV_W = HB / THRESH
CH = 16384

N_PIX = 8 * 512 * 512
PER_W = N_PIX // NW
N_CHUNKS = PER_W // CH
N_MIN = float(max(N_PIX // 16, 1))


def _tc_loss_body(p_ref, t_ref, o_ref):
    t = t_ref[0]
    m = p_ref[0, 0]
    for c in range(1, 19):
        m = jnp.maximum(m, p_ref[0, c])
    s = jnp.zeros_like(m)
    xt = jnp.zeros_like(m)
    for c in range(19):
        xc = p_ref[0, c]
        s = s + jnp.exp(xc - m)
        xt = jnp.where(t == c, xc, xt)
    loss = m + jnp.log(s) - xt
    o_ref[...] = jnp.where(t == IGNORE_LABEL, 0.0, loss).reshape(-1)


def _tc_loss(preds, targets, b0, nb):
    B, C, H, W = preds.shape
    return pl.pallas_call(
        _tc_loss_body,
        grid=(nb,),
        in_specs=[
            pl.BlockSpec((1, C, H, W), lambda b: (b0 + b, 0, 0, 0)),
            pl.BlockSpec((1, H, W), lambda b: (b0 + b, 0, 0)),
        ],
        out_specs=pl.BlockSpec((H * W,), lambda b: (b,)),
        out_shape=jax.ShapeDtypeStruct((nb * H * W,), jnp.float32),
        compiler_params=pltpu.CompilerParams(
            dimension_semantics=("arbitrary",),
        ),
    )(preds, targets)


def _sc_hard_body(n_chunks, loss_hbm, hard_out, buf0, buf1, obuf, sem0, sem1):
    wid = lax.axis_index("s") * NC + lax.axis_index("c")
    base = wid * (n_chunks * CH)
    bufs = (buf0, buf1)
    sems = (sem0, sem1)

    def copy(chunk):
        return pltpu.make_async_copy(
            loss_hbm.at[pl.ds(base + chunk * CH, CH)],
            bufs[chunk % 2], sems[chunk % 2])

    def gbody(buf):
        def body(g, carry):
            hc, hs = carry
            for u in range(4):
                v = buf[pl.ds((g * 4 + u) * L, L)]
                hard = v > THRESH
                hc = hc + jnp.where(hard, 1.0, 0.0)
                hs = hs + jnp.where(hard, v, 0.0)
            return hc, hs
        return body

    hc = jnp.zeros((L,), jnp.float32)
    hs = jnp.zeros((L,), jnp.float32)
    copy(0).start()
    for chunk in range(n_chunks):
        if chunk + 1 < n_chunks:
            copy(chunk + 1).start()
        copy(chunk).wait()
        hc, hs = lax.fori_loop(0, CH // (4 * L), gbody(bufs[chunk % 2]),
                               (hc, hs))

    obuf[0] = hc
    obuf[1] = hs
    pltpu.sync_copy(obuf, hard_out.at[wid])


def _sc_hard(loss_flat):
    n_chunks = loss_flat.shape[0] // (NW * CH)
    mesh = plsc.VectorSubcoreMesh(core_axis_name="c", subcore_axis_name="s")
    kern = pl.kernel(
        functools.partial(_sc_hard_body, n_chunks),
        mesh=mesh,
        out_type=jax.ShapeDtypeStruct((NW, 2, L), jnp.float32),
        scratch_types=[
            pltpu.VMEM((CH,), jnp.float32),
            pltpu.VMEM((CH,), jnp.float32),
            pltpu.VMEM((2, L), jnp.float32),
            pltpu.SemaphoreType.DMA,
            pltpu.SemaphoreType.DMA,
        ],
        compiler_params=pltpu.CompilerParams(needs_layout_passes=False),
    )
    return kern(loss_flat)


def _sc_mine_body(loss_hbm, cnt_out, sum_out, hard_out,
                  buf, cnts, sums, rcnt, rsum, hbuf):
    wid = lax.axis_index("s") * NC + lax.axis_index("c")
    base = wid * PER_W

    zero16 = jnp.zeros((L,), jnp.float32)

    def zbody(i, _):
        cnts[pl.ds(i * L, L)] = zero16
        sums[pl.ds(i * L, L)] = zero16
        return 0

    lax.fori_loop(0, HB, zbody, 0)

    lanes = lax.iota(jnp.int32, L) * HB
    ones = jnp.ones((L,), jnp.float32)

    def gbody(g, carry):
        hc, hs = carry
        v = buf[pl.ds(g * L, L)]
        hard = v > THRESH
        hc = hc + jnp.where(hard, 1.0, 0.0)
        hs = hs + jnp.where(hard, v, 0.0)
        b = (v * INV_W).astype(jnp.int32)
        b = jnp.minimum(b, HB - 1)
        fidx = lanes + b
        easy = jnp.logical_not(hard)
        plsc.addupdate_scatter(cnts, [fidx], ones, mask=easy)
        plsc.addupdate_scatter(sums, [fidx], v, mask=easy)
        return hc, hs

    hc, hs = zero16, zero16
    for chunk in range(N_CHUNKS):
        pltpu.sync_copy(loss_hbm.at[pl.ds(base + chunk * CH, CH)], buf)
        hc, hs = lax.fori_loop(0, CH // L, gbody, (hc, hs))

    def rbody(j, _):
        ac = zero16
        asm = zero16
        for l in range(L):
            ac = ac + cnts[pl.ds(l * HB + j * L, L)]
            asm = asm + sums[pl.ds(l * HB + j * L, L)]
        rcnt[pl.ds(j * L, L)] = ac
        rsum[pl.ds(j * L, L)] = asm
        return 0

    lax.fori_loop(0, HB // L, rbody, 0)

    hbuf[0] = hc
    hbuf[1] = hs
    pltpu.sync_copy(rcnt, cnt_out.at[wid])
    pltpu.sync_copy(rsum, sum_out.at[wid])
    pltpu.sync_copy(hbuf, hard_out.at[wid])


def _sc_mine(loss_flat):
    mesh = plsc.VectorSubcoreMesh(core_axis_name="c", subcore_axis_name="s")
    kern = pl.kernel(
        _sc_mine_body,
        mesh=mesh,
        out_type=[
            jax.ShapeDtypeStruct((NW, HB), jnp.float32),
            jax.ShapeDtypeStruct((NW, HB), jnp.float32),
            jax.ShapeDtypeStruct((NW, 2, L), jnp.float32),
        ],
        scratch_types=[
            pltpu.VMEM((CH,), jnp.float32),
            pltpu.VMEM((L * HB,), jnp.float32),
            pltpu.VMEM((L * HB,), jnp.float32),
            pltpu.VMEM((HB,), jnp.float32),
            pltpu.VMEM((HB,), jnp.float32),
            pltpu.VMEM((2, L), jnp.float32),
        ],
        compiler_params=pltpu.CompilerParams(needs_layout_passes=False),
    )
    return kern(loss_flat)


def _topk_mean(cnt_hist, sum_hist, hard):
    cnt_b = jnp.sum(cnt_hist, axis=0)
    sum_b = jnp.sum(sum_hist, axis=0)
    h = jnp.sum(hard, axis=(0, 2))
    n_hard, sum_hard = h[0], h[1]
    cc = jnp.cumsum(cnt_b[::-1])[::-1]
    above = cc - cnt_b
    need = N_MIN - n_hard
    r = jnp.clip(need - above, 0.0, cnt_b)
    bin_mean = sum_b / jnp.maximum(cnt_b, 1.0)
    return (sum_hard + jnp.sum(r * bin_mean)) / N_MIN


def _finish(cnt_hist, sum_hist, hard):
    h = jnp.sum(hard, axis=(0, 2))
    n_hard = h[0]
    hard_mean = h[1] / jnp.maximum(n_hard, 1.0)
    return jnp.where(n_hard < N_MIN, _topk_mean(cnt_hist, sum_hist, hard),
                     hard_mean)


S = 1
BS = 8 // S


def kernel(preds, targets):
    t32 = targets.astype(jnp.int32)
    parts = [_tc_loss(preds, t32, i * BS, BS) for i in range(S)]
    hstats = [_sc_hard(p) for p in parts]
    h = jnp.sum(jnp.stack(hstats), axis=(0, 1, 3))
    n_hard, sum_hard = h[0], h[1]
    hard_mean = sum_hard / jnp.maximum(n_hard, 1.0)

    def rare(_):
        cnt_hist, sum_hist, hard = _sc_mine(jnp.concatenate(parts))
        return _topk_mean(cnt_hist, sum_hist, hard)

    def common(_):
        return hard_mean

    return lax.cond(n_hard < N_MIN, rare, common, None)


# speedup vs baseline: 1.2049x; 1.0029x over previous
import functools

import jax
import jax.numpy as jnp
from jax import lax
from jax.experimental import pallas as pl
from jax.experimental.pallas import tpu as pltpu
from jax.experimental.pallas import tpu_sc as plsc

IGNORE_LABEL = 255
THRESH = 0.35667494393873245

NC, NS, L = 2, 16, 16
NW = NC * NS

HB = 1024
INV_W = HB / THRESH
CH = 16384
MCH = 8192

N_PIX = 8 * 512 * 512
PER_W = N_PIX // NW
M_CHUNKS = PER_W // MCH
N_MIN = float(max(N_PIX // 16, 1))


def _tc_loss_body(p_ref, t_ref, o_ref):
    t = t_ref[0]
    m = p_ref[0, 0]
    for c in range(1, 19):
        m = jnp.maximum(m, p_ref[0, c])
    s = jnp.zeros_like(m)
    xt = jnp.zeros_like(m)
    for c in range(19):
        xc = p_ref[0, c]
        s = s + jnp.exp(xc - m)
        xt = jnp.where(t == c, xc, xt)
    loss = m + jnp.log(s) - xt
    o_ref[...] = jnp.where(t == IGNORE_LABEL, 0.0, loss).reshape(-1)


def _tc_loss(preds, targets, b0, nb):
    B, C, H, W = preds.shape
    return pl.pallas_call(
        _tc_loss_body,
        grid=(nb,),
        in_specs=[
            pl.BlockSpec((1, C, H, W), lambda b: (b0 + b, 0, 0, 0)),
            pl.BlockSpec((1, H, W), lambda b: (b0 + b, 0, 0)),
        ],
        out_specs=pl.BlockSpec((H * W,), lambda b: (b,)),
        out_shape=jax.ShapeDtypeStruct((nb * H * W,), jnp.float32),
        compiler_params=pltpu.CompilerParams(
            dimension_semantics=("arbitrary",),
        ),
    )(preds, targets)


def _sc_hard_body(n_chunks, loss_hbm, hard_out, buf0, buf1, obuf, sem0, sem1):
    wid = lax.axis_index("s") * NC + lax.axis_index("c")
    base = wid * (n_chunks * CH)
    bufs = (buf0, buf1)
    sems = (sem0, sem1)

    def copy(chunk):
        return pltpu.make_async_copy(
            loss_hbm.at[pl.ds(base + chunk * CH, CH)],
            bufs[chunk % 2], sems[chunk % 2])

    def gbody(buf):
        def body(g, carry):
            hc, hs = carry
            for u in range(4):
                v = buf[pl.ds((g * 4 + u) * L, L)]
                hard = v > THRESH
                hc = hc + jnp.where(hard, 1.0, 0.0)
                hs = hs + jnp.where(hard, v, 0.0)
            return hc, hs
        return body

    hc = jnp.zeros((L,), jnp.float32)
    hs = jnp.zeros((L,), jnp.float32)
    copy(0).start()
    for chunk in range(n_chunks):
        if chunk + 1 < n_chunks:
            copy(chunk + 1).start()
        copy(chunk).wait()
        hc, hs = lax.fori_loop(0, CH // (4 * L), gbody(bufs[chunk % 2]),
                               (hc, hs))

    obuf[0] = hc
    obuf[1] = hs
    pltpu.sync_copy(obuf, hard_out.at[wid])


def _sc_hard(loss_flat):
    n_chunks = loss_flat.shape[0] // (NW * CH)
    mesh = plsc.VectorSubcoreMesh(core_axis_name="c", subcore_axis_name="s")
    kern = pl.kernel(
        functools.partial(_sc_hard_body, n_chunks),
        mesh=mesh,
        out_type=jax.ShapeDtypeStruct((NW, 2, L), jnp.float32),
        scratch_types=[
            pltpu.VMEM((CH,), jnp.float32),
            pltpu.VMEM((CH,), jnp.float32),
            pltpu.VMEM((2, L), jnp.float32),
            pltpu.SemaphoreType.DMA,
            pltpu.SemaphoreType.DMA,
        ],
        compiler_params=pltpu.CompilerParams(needs_layout_passes=False),
    )
    return kern(loss_flat)


def _sc_mine_body(loss_hbm, cnt_out, sum_out, hard_out,
                  buf, cnts, sums, rcnt, rsum, hbuf):
    wid = lax.axis_index("s") * NC + lax.axis_index("c")
    base = wid * PER_W

    zero16 = jnp.zeros((L,), jnp.float32)

    def zbody(i, _):
        cnts[pl.ds(i * L, L)] = zero16
        sums[pl.ds(i * L, L)] = zero16
        return 0

    lax.fori_loop(0, HB, zbody, 0)

    lanes = lax.iota(jnp.int32, L) * HB
    ones = jnp.ones((L,), jnp.float32)

    def gbody(g, carry):
        hc, hs = carry
        v = buf[pl.ds(g * L, L)]
        hard = v > THRESH
        hc = hc + jnp.where(hard, 1.0, 0.0)
        hs = hs + jnp.where(hard, v, 0.0)
        b = (v * INV_W).astype(jnp.int32)
        b = jnp.minimum(b, HB - 1)
        fidx = lanes + b
        easy = jnp.logical_not(hard)
        plsc.addupdate_scatter(cnts, [fidx], ones, mask=easy)
        plsc.addupdate_scatter(sums, [fidx], v, mask=easy)
        return hc, hs

    hc, hs = zero16, zero16
    for chunk in range(M_CHUNKS):
        pltpu.sync_copy(loss_hbm.at[pl.ds(base + chunk * MCH, MCH)], buf)
        hc, hs = lax.fori_loop(0, MCH // L, gbody, (hc, hs))

    def rbody(j, _):
        ac = zero16
        asm = zero16
        for l in range(L):
            ac = ac + cnts[pl.ds(l * HB + j * L, L)]
            asm = asm + sums[pl.ds(l * HB + j * L, L)]
        rcnt[pl.ds(j * L, L)] = ac
        rsum[pl.ds(j * L, L)] = asm
        return 0

    lax.fori_loop(0, HB // L, rbody, 0)

    hbuf[0] = hc
    hbuf[1] = hs
    pltpu.sync_copy(rcnt, cnt_out.at[wid])
    pltpu.sync_copy(rsum, sum_out.at[wid])
    pltpu.sync_copy(hbuf, hard_out.at[wid])


def _sc_mine(loss_flat):
    mesh = plsc.VectorSubcoreMesh(core_axis_name="c", subcore_axis_name="s")
    kern = pl.kernel(
        _sc_mine_body,
        mesh=mesh,
        out_type=[
            jax.ShapeDtypeStruct((NW, HB), jnp.float32),
            jax.ShapeDtypeStruct((NW, HB), jnp.float32),
            jax.ShapeDtypeStruct((NW, 2, L), jnp.float32),
        ],
        scratch_types=[
            pltpu.VMEM((MCH,), jnp.float32),
            pltpu.VMEM((L * HB,), jnp.float32),
            pltpu.VMEM((L * HB,), jnp.float32),
            pltpu.VMEM((HB,), jnp.float32),
            pltpu.VMEM((HB,), jnp.float32),
            pltpu.VMEM((2, L), jnp.float32),
        ],
        compiler_params=pltpu.CompilerParams(needs_layout_passes=False),
    )
    return kern(loss_flat)


def _topk_mean(cnt_hist, sum_hist, hard):
    cnt_b = jnp.sum(cnt_hist, axis=0)
    sum_b = jnp.sum(sum_hist, axis=0)
    h = jnp.sum(hard, axis=(0, 2))
    n_hard, sum_hard = h[0], h[1]
    cc = jnp.cumsum(cnt_b[::-1])[::-1]
    above = cc - cnt_b
    need = N_MIN - n_hard
    r = jnp.clip(need - above, 0.0, cnt_b)
    bin_mean = sum_b / jnp.maximum(cnt_b, 1.0)
    return (sum_hard + jnp.sum(r * bin_mean)) / N_MIN


def _finish(cnt_hist, sum_hist, hard):
    h = jnp.sum(hard, axis=(0, 2))
    n_hard = h[0]
    hard_mean = h[1] / jnp.maximum(n_hard, 1.0)
    return jnp.where(n_hard < N_MIN, _topk_mean(cnt_hist, sum_hist, hard),
                     hard_mean)


S = 1
BS = 8 // S


def kernel(preds, targets):
    t32 = targets.astype(jnp.int32)
    parts = [_tc_loss(preds, t32, i * BS, BS) for i in range(S)]
    hstats = [_sc_hard(p) for p in parts]
    h = jnp.sum(jnp.stack(hstats), axis=(0, 1, 3))
    n_hard, sum_hard = h[0], h[1]
    hard_mean = sum_hard / jnp.maximum(n_hard, 1.0)

    def rare(_):
        cnt_hist, sum_hist, hard = _sc_mine(jnp.concatenate(parts))
        return _topk_mean(cnt_hist, sum_hist, hard)

    def common(_):
        return hard_mean

    return lax.cond(n_hard < N_MIN, rare, common, None)
